# baseline (speedup 1.0000x reference)
import functools

import jax
import jax.numpy as jnp
from jax import lax
from jax.experimental import pallas as pl
from jax.experimental.pallas import tpu as pltpu
from jax.experimental.pallas import tpu_sc as plsc

_L = 16


def _sc_body(nw, nc, rows_w, ch, x_hbm, t_hbm, o_hbm, tbuf, xb0, xb1, xb2, xb3):
    wid = lax.axis_index("s") * nc + lax.axis_index("c")
    base = wid * rows_w
    xbs = (xb0, xb1, xb2, xb3)
    d = t_hbm.shape[1]

    def chunk(c, carry):
        row0 = base + c * ch
        pltpu.sync_copy(t_hbm.at[pl.ds(row0, ch)], tbuf)
        for b, xb in enumerate(xbs):
            pltpu.sync_copy(x_hbm.at[b, pl.ds(row0, ch)], xb)

        def row(r, rc):
            for j in range(d // _L):
                sl = pl.ds(j * _L, _L)
                t = tbuf[r, sl]
                for xb in xbs:
                    xb[r, sl] = xb[r, sl] + t
            return rc

        lax.fori_loop(0, ch, row, 0)
        for b, xb in enumerate(xbs):
            pltpu.sync_copy(xb, o_hbm.at[b, pl.ds(row0, ch)])
        return carry

    lax.fori_loop(0, rows_w // ch, chunk, 0)


def kernel(inputs, table):
    B, S, D = inputs.shape
    info = plsc.get_sparse_core_info()
    nc, ns = info.num_cores, info.num_subcores
    nw = nc * ns
    rows_w = S // nw
    ch = 8

    mesh = plsc.VectorSubcoreMesh(core_axis_name="c", subcore_axis_name="s")
    k = functools.partial(
        pl.kernel,
        mesh=mesh,
        out_type=jax.ShapeDtypeStruct((B, S, D), inputs.dtype),
        scratch_types=[pltpu.VMEM((ch, D), jnp.float32) for _ in range(5)],
    )(functools.partial(_sc_body, nw, nc, rows_w, ch))
    return k(inputs, table)


# speedup vs baseline: 1.8551x; 1.8551x over previous
import functools

import jax
import jax.numpy as jnp
from jax import lax
from jax.experimental import pallas as pl
from jax.experimental.pallas import tpu as pltpu
from jax.experimental.pallas import tpu_sc as plsc

_L = 16


def _sc_body(nc, rows_w, ch, x_hbm, t_hbm, o_hbm,
             tbA, xA0, xA1, xA2, xA3, tbB, xB0, xB1, xB2, xB3,
             isA, isB, osA, osB):
    wid = lax.axis_index("s") * nc + lax.axis_index("c")
    base = wid * rows_w
    nchunk = rows_w // ch
    d = t_hbm.shape[1]
    bufA = (tbA, xA0, xA1, xA2, xA3)
    bufB = (tbB, xB0, xB1, xB2, xB3)

    def fire_in(buf, isem, cidx):
        row0 = base + cidx * ch
        pltpu.async_copy(t_hbm.at[pl.ds(row0, ch)], buf[0], isem)
        for b in range(4):
            pltpu.async_copy(x_hbm.at[b, pl.ds(row0, ch)], buf[1 + b], isem)

    def wait_in(buf, isem):
        pltpu.make_async_copy(t_hbm.at[pl.ds(0, ch)], buf[0], isem).wait()
        for b in range(4):
            pltpu.make_async_copy(x_hbm.at[b, pl.ds(0, ch)], buf[1 + b], isem).wait()

    def fire_out(buf, osem, cidx):
        row0 = base + cidx * ch
        for b in range(4):
            pltpu.async_copy(buf[1 + b], o_hbm.at[b, pl.ds(row0, ch)], osem)

    def wait_out(buf, osem):
        for b in range(4):
            pltpu.make_async_copy(buf[1 + b], o_hbm.at[b, pl.ds(0, ch)], osem).wait()

    def compute(buf):
        tb = buf[0]

        def row(r, rc):
            for j in range(d // _L):
                sl = pl.ds(j * _L, _L)
                t = tb[r, sl]
                for b in range(4):
                    xb = buf[1 + b]
                    xb[r, sl] = xb[r, sl] + t
            return rc

        lax.fori_loop(0, ch, row, 0)

    fire_in(bufA, isA, 0)

    def body(c, carry):
        even = 2 * c
        fire_in(bufB, isB, even + 1)
        wait_in(bufA, isA)
        compute(bufA)
        fire_out(bufA, osA, even)
        wait_in(bufB, isB)
        compute(bufB)
        wait_out(bufA, osA)
        fire_in(bufA, isA, jnp.minimum(even + 2, nchunk - 2))
        fire_out(bufB, osB, even + 1)
        wait_out(bufB, osB)
        return carry

    lax.fori_loop(0, nchunk // 2, body, 0)
    wait_in(bufA, isA)


def kernel(inputs, table):
    B, S, D = inputs.shape
    info = plsc.get_sparse_core_info()
    nc, ns = info.num_cores, info.num_subcores
    nw = nc * ns
    rows_w = S // nw
    ch = 8

    mesh = plsc.VectorSubcoreMesh(core_axis_name="c", subcore_axis_name="s")
    k = functools.partial(
        pl.kernel,
        mesh=mesh,
        out_type=jax.ShapeDtypeStruct((B, S, D), inputs.dtype),
        scratch_types=(
            [pltpu.VMEM((ch, D), jnp.float32) for _ in range(10)]
            + [pltpu.SemaphoreType.DMA for _ in range(4)]
        ),
    )(functools.partial(_sc_body, nc, rows_w, ch))
    return k(inputs, table)
